# baseline (speedup 1.0000x reference)
import functools

import jax
import jax.numpy as jnp
from jax import lax
from jax.experimental import pallas as pl
from jax.experimental.pallas import tpu as pltpu
from jax.experimental.pallas import tpu_sc as plsc

BATCH = 1024
CTX = 200
HD = 64
NUM_CORES = 2
NUM_SUBCORES = 16
NW = NUM_CORES * NUM_SUBCORES
ROWS_PER_W = BATCH // NW
SPLIT = 120

_mesh = plsc.VectorSubcoreMesh(
    core_axis_name="c",
    subcore_axis_name="s",
    num_cores=NUM_CORES,
    num_subcores=NUM_SUBCORES,
)


def _emb_body(x_hbm, table_hbm, pos_hbm, out_hbm, idx_a, idx_b, rows_v, pos_v, sem):
    wid = lax.axis_index("s") * NUM_CORES + lax.axis_index("c")
    pltpu.sync_copy(pos_hbm, pos_v)
    base = wid * ROWS_PER_W * CTX

    for b in range(ROWS_PER_W):
        off = base + b * CTX
        pltpu.sync_copy(x_hbm.at[pl.ds(off, SPLIT)], idx_a)
        pltpu.sync_copy(x_hbm.at[pl.ds(off + SPLIT, CTX - SPLIT)], idx_b)
        cp0 = pltpu.async_copy(
            table_hbm.at[idx_a], rows_v.at[pl.ds(0, SPLIT)], sem
        )
        cp1 = pltpu.async_copy(
            table_hbm.at[idx_b], rows_v.at[pl.ds(SPLIT, CTX - SPLIT)], sem
        )
        cp0.wait()
        cp1.wait()

        def add_pos(j, carry):
            for c in range(HD // 16):
                plsc.addupdate(
                    rows_v.at[j, pl.ds(c * 16, 16)],
                    pos_v[j, pl.ds(c * 16, 16)],
                )
            return carry

        lax.fori_loop(0, CTX, add_pos, 0)
        pltpu.sync_copy(rows_v, out_hbm.at[pl.ds(off, CTX)])


@functools.partial(jax.jit, static_argnames=())
def _emb_call(x_flat, table, pos_encoding):
    return pl.kernel(
        _emb_body,
        out_type=jax.ShapeDtypeStruct((BATCH * CTX, HD), jnp.float32),
        mesh=_mesh,
        scratch_types=[
            pltpu.VMEM((SPLIT,), jnp.int32),
            pltpu.VMEM((CTX - SPLIT,), jnp.int32),
            pltpu.VMEM((CTX, HD), jnp.float32),
            pltpu.VMEM((CTX, HD), jnp.float32),
            pltpu.SemaphoreType.DMA,
        ],
        compiler_params=pltpu.CompilerParams(use_tc_tiling_on_sc=False),
    )(x_flat, table, pos_encoding)


def kernel(x, table, pos_encoding):
    x_flat = x.reshape(-1).astype(jnp.int32)
    out = _emb_call(x_flat, table, pos_encoding)
    return out.reshape(BATCH, CTX, HD)


# speedup vs baseline: 1.0831x; 1.0831x over previous
import functools

import jax
import jax.numpy as jnp
from jax import lax
from jax.experimental import pallas as pl
from jax.experimental.pallas import tpu as pltpu
from jax.experimental.pallas import tpu_sc as plsc

BATCH = 1024
CTX = 200
HD = 64
NUM_CORES = 2
NUM_SUBCORES = 16
NW = NUM_CORES * NUM_SUBCORES
ROWS_PER_W = BATCH // NW
R_STEP = 2
C_STEP = R_STEP * CTX
N_STEP = ROWS_PER_W // R_STEP
CHUNKS = (104, 104, 104, 88)

_mesh = plsc.VectorSubcoreMesh(
    core_axis_name="c",
    subcore_axis_name="s",
    num_cores=NUM_CORES,
    num_subcores=NUM_SUBCORES,
)


def _emb_body(x_hbm, table_hbm, pos_hbm, out_hbm, idx_v, rows_v, pos_v, gsem, osem):
    wid = lax.axis_index("s") * NUM_CORES + lax.axis_index("c")
    base = wid * ROWS_PER_W * CTX
    pltpu.sync_copy(x_hbm.at[pl.ds(base, ROWS_PER_W * CTX)], idx_v)
    pltpu.sync_copy(pos_hbm, pos_v)

    def start_gathers(s):
        p = s % 2
        cps = []
        o = 0
        for n in CHUNKS:
            cps.append(
                pltpu.async_copy(
                    table_hbm.at[idx_v.at[pl.ds(s * C_STEP + o, n)]],
                    rows_v.at[p, pl.ds(o, n)],
                    gsem.at[p],
                )
            )
            o += n
        return cps

    out_cp = [None, None]
    cps_cur = start_gathers(0)
    for s in range(N_STEP):
        p = s % 2
        if s + 1 < N_STEP:
            q = (s + 1) % 2
            if out_cp[q] is not None:
                out_cp[q].wait()
                out_cp[q] = None
            cps_next = start_gathers(s + 1)
        else:
            cps_next = None
        for cp in cps_cur:
            cp.wait()

        for r in range(R_STEP):
            def add_pos(j, carry):
                for c in range(HD // 16):
                    plsc.addupdate(
                        rows_v.at[p, r * CTX + j, pl.ds(c * 16, 16)],
                        pos_v[j, pl.ds(c * 16, 16)],
                    )
                return carry

            lax.fori_loop(0, CTX, add_pos, 0)

        out_cp[p] = pltpu.async_copy(
            rows_v.at[p],
            out_hbm.at[pl.ds(base + s * C_STEP, C_STEP)],
            osem.at[p],
        )
        cps_cur = cps_next

    for cp in out_cp:
        if cp is not None:
            cp.wait()


@functools.partial(jax.jit, static_argnames=())
def _emb_call(x_flat, table, pos_encoding):
    return pl.kernel(
        _emb_body,
        out_type=jax.ShapeDtypeStruct((BATCH * CTX, HD), jnp.float32),
        mesh=_mesh,
        scratch_types=[
            pltpu.VMEM((ROWS_PER_W * CTX,), jnp.int32),
            pltpu.VMEM((2, C_STEP, HD), jnp.float32),
            pltpu.VMEM((CTX, HD), jnp.float32),
            pltpu.SemaphoreType.DMA((2,)),
            pltpu.SemaphoreType.DMA((2,)),
        ],
        compiler_params=pltpu.CompilerParams(use_tc_tiling_on_sc=False),
    )(x_flat, table, pos_encoding)


def kernel(x, table, pos_encoding):
    x_flat = x.reshape(-1).astype(jnp.int32)
    out = _emb_call(x_flat, table, pos_encoding)
    return out.reshape(BATCH, CTX, HD)
